# baseline (speedup 1.0000x reference)
import functools

import jax
import jax.numpy as jnp
from jax import lax
from jax.experimental import pallas as pl
from jax.experimental.pallas import tpu as pltpu
from jax.experimental.pallas import tpu_sc as plsc

_NC = 2
_NS = 16
_L = 16
_CK = 16
_NBUF = 4
_JB = 16


def _pe_add_body(x_hbm, pe_hbm, out_hbm, xb, peb,
                 ls0, ls1, ls2, ls3, ss0, ss1, ss2, ss3, ps0, ps1):
    B = out_hbm.shape[0]
    S = out_hbm.shape[1]
    D = out_hbm.shape[2]
    nw = _NC * _NS
    sw = S // nw
    nk = sw // _CK
    wid = lax.axis_index("s") * _NC + lax.axis_index("c")
    s_base = wid * sw
    lsems = [ls0, ls1, ls2, ls3]
    ssems = [ss0, ss1, ss2, ss3]
    psems = [ps0, ps1]

    def issue_xload(k, b, u):
        s0 = s_base + k * _CK
        pltpu.async_copy(x_hbm.at[b, pl.ds(s0, _CK)], xb.at[u], lsems[u])

    def wait_xload(u):
        pltpu.make_async_copy(x_hbm.at[0, pl.ds(0, _CK)], xb.at[u],
                              lsems[u]).wait()

    def issue_peload(k, kp):
        s0 = s_base + k * _CK
        pltpu.async_copy(pe_hbm.at[pl.ds(s0, _CK)], peb.at[kp], psems[kp])

    def wait_peload(kp):
        pltpu.make_async_copy(pe_hbm.at[pl.ds(0, _CK)], peb.at[kp],
                              psems[kp]).wait()

    def issue_store(k, b, u):
        s0 = s_base + k * _CK
        pltpu.async_copy(xb.at[u], out_hbm.at[b, pl.ds(s0, _CK)], ssems[u])

    def wait_store(u):
        pltpu.make_async_copy(xb.at[u], out_hbm.at[0, pl.ds(0, _CK)],
                              ssems[u]).wait()

    def compute(u, kp):
        def row_body(r, _):
            for jb in range(D // (_JB * _L)):
                base = jb * _JB * _L
                for i in range(_JB):
                    off = base + i * _L
                    xb[u, r, pl.ds(off, _L)] = (
                        xb[u, r, pl.ds(off, _L)] + peb[kp, r, pl.ds(off, _L)]
                    )
            return 0

        lax.fori_loop(0, _CK, row_body, 0)

    issue_peload(0, 0)
    issue_peload(1, 1)
    issue_xload(0, 0, 0)
    issue_xload(0, 1, 1)

    def outer(k2, _):
        for kp in range(2):
            k = k2 * 2 + kp
            wait_peload(kp)
            for u in range(B):
                pu = (u + 2) % _NBUF
                if u < 2:
                    @pl.when(k > 0)
                    def _():
                        wait_store(pu)
                    issue_xload(k, u + 2, pu)
                else:
                    wait_store(pu)

                    @pl.when(k < nk - 1)
                    def _():
                        issue_xload(k + 1, u - 2, pu)
                wait_xload(u)
                compute(u, kp)
                issue_store(k, u, u)
            @pl.when(k < nk - 2)
            def _():
                issue_peload(k + 2, kp)
        return 0

    lax.fori_loop(0, nk // 2, outer, 0)
    wait_store(2)
    wait_store(3)


def kernel(x, pe):
    B, S, D = x.shape
    mesh = plsc.VectorSubcoreMesh(core_axis_name="c", subcore_axis_name="s")
    run = functools.partial(
        pl.kernel,
        mesh=mesh,
        out_type=jax.ShapeDtypeStruct((B, S, D), x.dtype),
        scratch_types=[
            pltpu.VMEM((_NBUF, _CK, D), jnp.float32),
            pltpu.VMEM((2, _CK, D), jnp.float32),
            pltpu.SemaphoreType.DMA,
            pltpu.SemaphoreType.DMA,
            pltpu.SemaphoreType.DMA,
            pltpu.SemaphoreType.DMA,
            pltpu.SemaphoreType.DMA,
            pltpu.SemaphoreType.DMA,
            pltpu.SemaphoreType.DMA,
            pltpu.SemaphoreType.DMA,
            pltpu.SemaphoreType.DMA,
            pltpu.SemaphoreType.DMA,
        ],
    )(_pe_add_body)
    return run(x, pe)


# speedup vs baseline: 1.6386x; 1.6386x over previous
import functools

import jax
import jax.numpy as jnp
from jax import lax
from jax.experimental import pallas as pl
from jax.experimental.pallas import tpu as pltpu
from jax.experimental.pallas import tpu_sc as plsc

_NC = 2
_NS = 16
_L = 16
_C = 4
_NBUF = 4
_JB = 16


def _pe_add_body(x_hbm, pe_hbm, out_hbm, xb, peb,
                 ls0, ls1, ls2, ls3, ss0, ss1, ss2, ss3):
    B = out_hbm.shape[0]
    S = out_hbm.shape[1]
    D = out_hbm.shape[2]
    nw = _NC * _NS
    sw = S // nw
    n_steps = sw // _C
    n_outer = n_steps // _NBUF
    wid = lax.axis_index("s") * _NC + lax.axis_index("c")
    s_base = wid * sw
    lsems = [ls0, ls1, ls2, ls3]
    ssems = [ss0, ss1, ss2, ss3]

    def issue_loads(step, u):
        s0 = s_base + step * _C
        pltpu.async_copy(x_hbm.at[:, pl.ds(s0, _C)], xb.at[u], lsems[u])
        pltpu.async_copy(pe_hbm.at[pl.ds(s0, _C)], peb.at[u], lsems[u])

    def wait_loads(u):
        pltpu.make_async_copy(x_hbm.at[:, pl.ds(0, _C)], xb.at[u],
                              lsems[u]).wait()
        pltpu.make_async_copy(pe_hbm.at[pl.ds(0, _C)], peb.at[u],
                              lsems[u]).wait()

    def issue_store(step, u):
        s0 = s_base + step * _C
        pltpu.async_copy(xb.at[u], out_hbm.at[:, pl.ds(s0, _C)], ssems[u])

    def wait_store(u):
        pltpu.make_async_copy(xb.at[u], out_hbm.at[:, pl.ds(0, _C)],
                              ssems[u]).wait()

    def compute(u):
        def row_body(r, _):
            for jb in range(D // (_JB * _L)):
                base = jb * _JB * _L
                pe_vs = [peb[u, r, pl.ds(base + i * _L, _L)]
                         for i in range(_JB)]
                for b in range(B):
                    for i in range(_JB):
                        off = base + i * _L
                        xb[u, b, r, pl.ds(off, _L)] = (
                            xb[u, b, r, pl.ds(off, _L)] + pe_vs[i]
                        )
            return 0

        lax.fori_loop(0, _C, row_body, 0)

    issue_loads(0, 0)
    issue_loads(1, 1)

    def outer(kk4, _):
        for u in range(_NBUF):
            kk = kk4 * _NBUF + u
            pu = (u + 2) % _NBUF
            if u < 2:
                @pl.when(kk4 > 0)
                def _():
                    wait_store(pu)
                issue_loads(kk + 2, pu)
            else:
                wait_store(pu)

                @pl.when(kk4 < n_outer - 1)
                def _():
                    issue_loads(kk + 2, pu)
            wait_loads(u)
            compute(u)
            issue_store(kk, u)
        return 0

    lax.fori_loop(0, n_outer, outer, 0)
    wait_store(2)
    wait_store(3)


def kernel(x, pe):
    B, S, D = x.shape
    mesh = plsc.VectorSubcoreMesh(core_axis_name="c", subcore_axis_name="s")
    run = functools.partial(
        pl.kernel,
        mesh=mesh,
        out_type=jax.ShapeDtypeStruct((B, S, D), x.dtype),
        scratch_types=[
            pltpu.VMEM((_NBUF, B, _C, D), jnp.float32),
            pltpu.VMEM((_NBUF, _C, D), jnp.float32),
            pltpu.SemaphoreType.DMA,
            pltpu.SemaphoreType.DMA,
            pltpu.SemaphoreType.DMA,
            pltpu.SemaphoreType.DMA,
            pltpu.SemaphoreType.DMA,
            pltpu.SemaphoreType.DMA,
            pltpu.SemaphoreType.DMA,
            pltpu.SemaphoreType.DMA,
        ],
    )(_pe_add_body)
    return run(x, pe)
